# Initial kernel scaffold; baseline (speedup 1.0000x reference)
import jax
import jax.numpy as jnp
from jax.experimental import pallas as pl


def kernel(z, edge_index, W1, b1, W2, b2):
    raise NotImplementedError("write your pallas kernel here")


# speedup vs baseline: 18.6054x; 18.6054x over previous
import functools

import jax
import jax.numpy as jnp
from jax import lax
from jax.experimental import pallas as pl
from jax.experimental.pallas import tpu as pltpu
from jax.experimental.pallas import tpu_sc as plsc

N = 10000
E = 320000
D = 128

NC = 2
NS = 16
NW = NC * NS

CHUNK = 80
NCHUNK = E // (NW * CHUNK)
ROWS_PT = N // NS

DEG_W = 8

RBLK = 2000
NBLK = N // RBLK

_mesh = plsc.VectorSubcoreMesh(core_axis_name="c", subcore_axis_name="s")


@functools.partial(
    pl.kernel,
    out_type=jax.ShapeDtypeStruct((NC, NS, ROWS_PT, DEG_W), jnp.float32),
    mesh=_mesh,
    scratch_types=[
        pltpu.VMEM((NCHUNK, CHUNK), jnp.int32),
        pltpu.VMEM((CHUNK, DEG_W), jnp.float32),
        pltpu.VMEM_SHARED((N, DEG_W), jnp.float32),
    ],
)
def _deg_kernel(dst_hbm, zeros_hbm, ones_hbm, degp_hbm, dst_v, ones_v, acc_sh):
    c = lax.axis_index("c")
    s = lax.axis_index("s")
    w = c * NS + s
    pltpu.sync_copy(zeros_hbm.at[s], acc_sh.at[pl.ds(s * ROWS_PT, ROWS_PT)])
    pltpu.sync_copy(dst_hbm.at[w], dst_v)
    pltpu.sync_copy(ones_hbm, ones_v)
    plsc.subcore_barrier()

    def step(j, carry):
        pltpu.sync_copy(ones_v, acc_sh.at[dst_v.at[j]], add=True)
        return carry

    lax.fori_loop(0, NCHUNK, step, 0)
    plsc.subcore_barrier()
    pltpu.sync_copy(acc_sh.at[pl.ds(s * ROWS_PT, ROWS_PT)], degp_hbm.at[c].at[s])


@functools.partial(
    pl.kernel,
    out_type=jax.ShapeDtypeStruct((NC, NS, ROWS_PT, D), jnp.float32),
    mesh=_mesh,
    scratch_types=[
        pltpu.VMEM((NCHUNK, CHUNK), jnp.int32),
        pltpu.VMEM((NCHUNK, CHUNK), jnp.int32),
        pltpu.VMEM((CHUNK, D), jnp.float32),
        pltpu.VMEM_SHARED((N, D), jnp.float32),
        pltpu.SemaphoreType.DMA,
    ],
)
def _scatter_kernel(y_hbm, src_hbm, dst_hbm, zeros_hbm, part_hbm,
                    src_v, dst_v, rows_v, acc_sh, sem):
    c = lax.axis_index("c")
    s = lax.axis_index("s")
    w = c * NS + s
    pltpu.sync_copy(zeros_hbm.at[s], acc_sh.at[pl.ds(s * ROWS_PT, ROWS_PT)])
    pltpu.sync_copy(src_hbm.at[w], src_v)
    pltpu.sync_copy(dst_hbm.at[w], dst_v)
    plsc.subcore_barrier()

    def step(j, carry):
        pltpu.async_copy(y_hbm.at[src_v.at[j]], rows_v, sem).wait()
        pltpu.sync_copy(rows_v, acc_sh.at[dst_v.at[j]], add=True)
        return carry

    lax.fori_loop(0, NCHUNK, step, 0)
    plsc.subcore_barrier()
    pltpu.sync_copy(acc_sh.at[pl.ds(s * ROWS_PT, ROWS_PT)], part_hbm.at[c].at[s])


def _dinv_from(degp_ref):
    deg = degp_ref[0, :, 0:1] + degp_ref[1, :, 0:1] + 1.0
    return lax.rsqrt(deg)


def _tc1_body(z_ref, w1_ref, degp_ref, y1_ref):
    dinv = _dinv_from(degp_ref)
    xw = jnp.dot(z_ref[...], w1_ref[...], preferred_element_type=jnp.float32)
    y1_ref[...] = xw * dinv


def _tc2_body(p_ref, y1_ref, degp_ref, b1_ref, w2_ref, y2_ref):
    dinv = _dinv_from(degp_ref)
    h = dinv * (p_ref[0] + p_ref[1] + y1_ref[...]) + b1_ref[...]
    h = jnp.where(h > 0, h, 0.01 * h)
    y2_ref[...] = jnp.dot(h, w2_ref[...], preferred_element_type=jnp.float32) * dinv


def _tc3_body(p_ref, y2_ref, degp_ref, b2_ref, out_ref):
    dinv = _dinv_from(degp_ref)
    out_ref[...] = dinv * (p_ref[0] + p_ref[1] + y2_ref[...]) + b2_ref[...]


_blk_nd = pl.BlockSpec((RBLK, D), lambda i: (i, 0))
_blk_pnd = pl.BlockSpec((NC, RBLK, D), lambda i: (0, i, 0))
_blk_deg = pl.BlockSpec((NC, RBLK, DEG_W), lambda i: (0, i, 0))
_blk_w = pl.BlockSpec((D, D), lambda i: (0, 0))
_blk_b = pl.BlockSpec((1, D), lambda i: (0, 0))

_tc1 = pl.pallas_call(
    _tc1_body,
    grid=(NBLK,),
    in_specs=[_blk_nd, _blk_w, _blk_deg],
    out_specs=_blk_nd,
    out_shape=jax.ShapeDtypeStruct((N, D), jnp.float32),
)
_tc2 = pl.pallas_call(
    _tc2_body,
    grid=(NBLK,),
    in_specs=[_blk_pnd, _blk_nd, _blk_deg, _blk_b, _blk_w],
    out_specs=_blk_nd,
    out_shape=jax.ShapeDtypeStruct((N, D), jnp.float32),
)
_tc3 = pl.pallas_call(
    _tc3_body,
    grid=(NBLK,),
    in_specs=[_blk_pnd, _blk_nd, _blk_deg, _blk_b],
    out_specs=_blk_nd,
    out_shape=jax.ShapeDtypeStruct((N, D), jnp.float32),
)


def kernel(z, edge_index, W1, b1, W2, b2):
    src3d = edge_index[0].reshape(NW, NCHUNK, CHUNK)
    dst3d = edge_index[1].reshape(NW, NCHUNK, CHUNK)
    zeros_nd = jnp.zeros((NS, ROWS_PT, D), jnp.float32)
    zeros_nw = jnp.zeros((NS, ROWS_PT, DEG_W), jnp.float32)
    ones_cw = jnp.ones((CHUNK, DEG_W), jnp.float32)
    b1r = b1.reshape(1, D)
    b2r = b2.reshape(1, D)

    degp = _deg_kernel(dst3d, zeros_nw, ones_cw).reshape(NC, N, DEG_W)
    y1 = _tc1(z, W1, degp)
    p1 = _scatter_kernel(y1, src3d, dst3d, zeros_nd).reshape(NC, N, D)
    y2 = _tc2(p1, y1, degp, b1r, W2)
    p2 = _scatter_kernel(y2, src3d, dst3d, zeros_nd).reshape(NC, N, D)
    return _tc3(p2, y2, degp, b2r)
